# baseline (speedup 1.0000x reference)
import functools

import jax
import jax.numpy as jnp
from jax import lax
from jax.experimental import pallas as pl
from jax.experimental.pallas import tpu as pltpu
from jax.experimental.pallas import tpu_sc as plsc

F = 100000
L = 16
UNROLL = 4


@functools.lru_cache(maxsize=None)
def _build_sc_kernel():
    info = plsc.get_sparse_core_info()
    ns = info.num_subcores
    nc = 1
    nw = nc * ns
    step = L * UNROLL
    chunk = ((F + nw - 1) // nw + step - 1) // step * step
    n_outer = chunk // step
    mesh = plsc.VectorSubcoreMesh(core_axis_name="c", subcore_axis_name="s",
                                  num_cores=nc, num_subcores=ns)

    @functools.partial(
        pl.kernel,
        out_type=jax.ShapeDtypeStruct((F,), jnp.float32),
        mesh=mesh,
        scratch_types=[
            pltpu.VMEM((chunk,), jnp.float32),
            pltpu.VMEM((chunk,), jnp.float32),
            pltpu.VMEM((chunk,), jnp.float32),
            pltpu.VMEM((chunk,), jnp.float32),
            pltpu.VMEM((4 * L,), jnp.float32),
            pltpu.SemaphoreType.DMA,
        ],
    )
    def sc_kernel(x0_hbm, x1_hbm, x2_hbm, params_hbm, out_hbm,
                  x0_v, x1_v, x2_v, o_v, p_v, sem):
        wid = lax.axis_index("s") * nc + lax.axis_index("c")
        base = jnp.minimum(wid * chunk, F - chunk)
        c0 = pltpu.async_copy(params_hbm, p_v, sem)
        c1 = pltpu.async_copy(x0_hbm.at[pl.ds(base, chunk)], x0_v, sem)
        c2 = pltpu.async_copy(x1_hbm.at[pl.ds(base, chunk)], x1_v, sem)
        c3 = pltpu.async_copy(x2_hbm.at[pl.ds(base, chunk)], x2_v, sem)
        c0.wait()
        c1.wait()
        c2.wait()
        c3.wait()
        w0 = p_v[pl.ds(0 * L, L)]
        w1 = p_v[pl.ds(1 * L, L)]
        w2 = p_v[pl.ds(2 * L, L)]
        cc = p_v[pl.ds(3 * L, L)]

        def body(i, _):
            for j in range(UNROLL):
                off = (i * UNROLL + j) * L
                a0 = x0_v[pl.ds(off, L)]
                a1 = x1_v[pl.ds(off, L)]
                a2 = x2_v[pl.ds(off, L)]
                s = w0 * a0 + w1 * a1 + w2 * a2 + cc
                alpha = 1.0 / (1.0 + jnp.exp(-s))
                o_v[pl.ds(off, L)] = alpha * (a0 + a1 + a2)
            return _

        lax.fori_loop(0, n_outer, body, None)
        pltpu.sync_copy(o_v, out_hbm.at[pl.ds(base, chunk)])

    return sc_kernel


def kernel(intensities, avg_dist, W_gcn, b_gcn, W_attn, b_attn):
    del W_gcn, b_gcn
    w = W_attn[0]
    c = w[3] * avg_dist + b_attn[0]
    params = jnp.concatenate([
        jnp.full((L,), w[0], jnp.float32),
        jnp.full((L,), w[1], jnp.float32),
        jnp.full((L,), w[2], jnp.float32),
        jnp.full((L,), c, jnp.float32),
    ])
    return _build_sc_kernel()(intensities[0], intensities[1], intensities[2],
                              params)


# speedup vs baseline: 1.0080x; 1.0080x over previous
import functools

import jax
import jax.numpy as jnp
from jax import lax
from jax.experimental import pallas as pl
from jax.experimental.pallas import tpu as pltpu
from jax.experimental.pallas import tpu_sc as plsc

F = 100000
L = 16
UNROLL = 8


@functools.lru_cache(maxsize=None)
def _build_sc_kernel():
    info = plsc.get_sparse_core_info()
    ns = info.num_subcores
    nc = 1
    nw = nc * ns
    step = L * UNROLL
    chunk = ((F + nw - 1) // nw + step - 1) // step * step
    n_outer = chunk // step
    mesh = plsc.VectorSubcoreMesh(core_axis_name="c", subcore_axis_name="s",
                                  num_cores=nc, num_subcores=ns)

    @functools.partial(
        pl.kernel,
        out_type=jax.ShapeDtypeStruct((F,), jnp.float32),
        mesh=mesh,
        scratch_types=[
            pltpu.VMEM((chunk,), jnp.float32),
            pltpu.VMEM((chunk,), jnp.float32),
            pltpu.VMEM((chunk,), jnp.float32),
            pltpu.VMEM((chunk,), jnp.float32),
            pltpu.VMEM((4 * L,), jnp.float32),
            pltpu.SemaphoreType.DMA,
        ],
    )
    def sc_kernel(x0_hbm, x1_hbm, x2_hbm, params_hbm, out_hbm,
                  x0_v, x1_v, x2_v, o_v, p_v, sem):
        wid = lax.axis_index("s") * nc + lax.axis_index("c")
        base = jnp.minimum(wid * chunk, F - chunk)
        c0 = pltpu.async_copy(params_hbm, p_v, sem)
        c1 = pltpu.async_copy(x0_hbm.at[pl.ds(base, chunk)], x0_v, sem)
        c2 = pltpu.async_copy(x1_hbm.at[pl.ds(base, chunk)], x1_v, sem)
        c3 = pltpu.async_copy(x2_hbm.at[pl.ds(base, chunk)], x2_v, sem)
        c0.wait()
        c1.wait()
        c2.wait()
        c3.wait()
        w0 = p_v[pl.ds(0 * L, L)]
        w1 = p_v[pl.ds(1 * L, L)]
        w2 = p_v[pl.ds(2 * L, L)]
        cc = p_v[pl.ds(3 * L, L)]

        def body(i, _):
            for j in range(UNROLL):
                off = (i * UNROLL + j) * L
                a0 = x0_v[pl.ds(off, L)]
                a1 = x1_v[pl.ds(off, L)]
                a2 = x2_v[pl.ds(off, L)]
                s = w0 * a0 + w1 * a1 + w2 * a2 + cc
                o_v[pl.ds(off, L)] = (a0 + a1 + a2) / (1.0 + jnp.exp(-s))
            return _

        lax.fori_loop(0, n_outer, body, None)
        pltpu.sync_copy(o_v, out_hbm.at[pl.ds(base, chunk)])

    return sc_kernel


def kernel(intensities, avg_dist, W_gcn, b_gcn, W_attn, b_attn):
    del W_gcn, b_gcn
    w = W_attn[0]
    c = w[3] * avg_dist + b_attn[0]
    params = jnp.concatenate([
        jnp.full((L,), w[0], jnp.float32),
        jnp.full((L,), w[1], jnp.float32),
        jnp.full((L,), w[2], jnp.float32),
        jnp.full((L,), c, jnp.float32),
    ])
    return _build_sc_kernel()(intensities[0], intensities[1], intensities[2],
                              params)


# speedup vs baseline: 1.0091x; 1.0011x over previous
import functools

import jax
import jax.numpy as jnp
from jax import lax
from jax.experimental import pallas as pl
from jax.experimental.pallas import tpu as pltpu
from jax.experimental.pallas import tpu_sc as plsc

F = 100000
L = 16
UNROLL = 8


@functools.lru_cache(maxsize=None)
def _build_sc_kernel():
    info = plsc.get_sparse_core_info()
    ns = info.num_subcores
    nc = 1
    nw = nc * ns
    step = L * UNROLL
    chunk = ((F + nw - 1) // nw + step - 1) // step * step
    n_outer = chunk // step
    mesh = plsc.VectorSubcoreMesh(core_axis_name="c", subcore_axis_name="s",
                                  num_cores=nc, num_subcores=ns)

    @functools.partial(
        pl.kernel,
        out_type=jax.ShapeDtypeStruct((F,), jnp.float32),
        mesh=mesh,
        scratch_types=[
            pltpu.VMEM((chunk,), jnp.float32),
            pltpu.VMEM((chunk,), jnp.float32),
            pltpu.VMEM((chunk,), jnp.float32),
            pltpu.VMEM((chunk,), jnp.float32),
            pltpu.VMEM((4 * L,), jnp.float32),
            pltpu.SemaphoreType.DMA,
            pltpu.SemaphoreType.DMA,
            pltpu.SemaphoreType.DMA,
        ],
    )
    def sc_kernel(x0_hbm, x1_hbm, x2_hbm, params_hbm, out_hbm,
                  x0_v, x1_v, x2_v, o_v, p_v, s0, s1, s2):
        wid = lax.axis_index("s") * nc + lax.axis_index("c")
        base = jnp.minimum(wid * chunk, F - chunk)
        half = chunk // 2
        cp = pltpu.async_copy(params_hbm, p_v, s0)
        h0 = [pltpu.async_copy(x_hbm.at[pl.ds(base, half)],
                               x_v.at[pl.ds(0, half)], s0)
              for x_hbm, x_v in ((x0_hbm, x0_v), (x1_hbm, x1_v), (x2_hbm, x2_v))]
        h1 = [pltpu.async_copy(x_hbm.at[pl.ds(base + half, half)],
                               x_v.at[pl.ds(half, half)], s1)
              for x_hbm, x_v in ((x0_hbm, x0_v), (x1_hbm, x1_v), (x2_hbm, x2_v))]
        cp.wait()
        for c in h0:
            c.wait()
        w0 = p_v[pl.ds(0 * L, L)]
        w1 = p_v[pl.ds(1 * L, L)]
        w2 = p_v[pl.ds(2 * L, L)]
        cc = p_v[pl.ds(3 * L, L)]

        def body(i, _):
            for j in range(UNROLL):
                off = (i * UNROLL + j) * L
                a0 = x0_v[pl.ds(off, L)]
                a1 = x1_v[pl.ds(off, L)]
                a2 = x2_v[pl.ds(off, L)]
                s = w0 * a0 + w1 * a1 + w2 * a2 + cc
                o_v[pl.ds(off, L)] = (a0 + a1 + a2) / (1.0 + jnp.exp(-s))
            return _

        lax.fori_loop(0, n_outer // 2, body, None)
        o0 = pltpu.async_copy(o_v.at[pl.ds(0, half)],
                              out_hbm.at[pl.ds(base, half)], s2)
        for c in h1:
            c.wait()
        lax.fori_loop(n_outer // 2, n_outer, body, None)
        o1 = pltpu.async_copy(o_v.at[pl.ds(half, half)],
                              out_hbm.at[pl.ds(base + half, half)], s2)
        o0.wait()
        o1.wait()

    return sc_kernel


def kernel(intensities, avg_dist, W_gcn, b_gcn, W_attn, b_attn):
    del W_gcn, b_gcn
    w = W_attn[0]
    c = w[3] * avg_dist + b_attn[0]
    params = jnp.concatenate([
        jnp.full((L,), w[0], jnp.float32),
        jnp.full((L,), w[1], jnp.float32),
        jnp.full((L,), w[2], jnp.float32),
        jnp.full((L,), c, jnp.float32),
    ])
    return _build_sc_kernel()(intensities[0], intensities[1], intensities[2],
                              params)
